# baseline (speedup 1.0000x reference)
import functools

import jax
import jax.numpy as jnp
from jax import lax
from jax.experimental import pallas as pl
from jax.experimental.pallas import tpu as pltpu
from jax.experimental.pallas import tpu_sc as plsc

VOCAB = 1000000
DIM = 64
R, A, M = 64, 4, 128
RA = R * A
NSRC = RA * M

NC, NS = 2, 16
NW = NC * NS
SPW = NSRC // NW // M
APW = RA // NW

PAIRS_PER_STEP = 8
STEPS = RA // PAIRS_PER_STEP

@functools.cache
def _sc_gather_fn():
    mesh = plsc.VectorSubcoreMesh(core_axis_name="c", subcore_axis_name="s")

    @functools.partial(
        pl.kernel,
        mesh=mesh,
        out_type=[
            jax.ShapeDtypeStruct((NW, SPW, M, DIM), jnp.float32),
            jax.ShapeDtypeStruct((NW, APW, DIM), jnp.float32),
        ],
        scratch_types=[
            pltpu.VMEM((SPW, M), jnp.int32),
            pltpu.VMEM((SPW, M, DIM), jnp.float32),
            pltpu.VMEM((APW,), jnp.int32),
            pltpu.VMEM((APW, DIM), jnp.float32),
            pltpu.SemaphoreType.DMA,
        ],
        compiler_params=pltpu.CompilerParams(use_tc_tiling_on_sc=False),
    )
    def _sc_gather(table_hbm, sidx_hbm, aidx_hbm, src_out, aux_out,
                   sidx_v, srows_v, aidx_v, arows_v, sem):
        wid = lax.axis_index("s") * NC + lax.axis_index("c")
        pltpu.sync_copy(sidx_hbm.at[wid], sidx_v)
        pltpu.sync_copy(aidx_hbm.at[wid], aidx_v)
        copies = [
            pltpu.async_copy(table_hbm.at[sidx_v.at[j]], srows_v.at[j], sem)
            for j in range(SPW)
        ]
        copies.append(pltpu.async_copy(table_hbm.at[aidx_v], arows_v, sem))
        for c in copies:
            c.wait()
        pltpu.sync_copy(srows_v, src_out.at[wid])
        pltpu.sync_copy(arows_v, aux_out.at[wid])

    return _sc_gather


def _tc_body(src_ref, aux_ref, out_ref, acc_ref):
    i = pl.program_id(0)

    @pl.when(i == 0)
    def _init():
        acc_ref[0] = 0.0
        acc_ref[1] = 0.0

    aux_blk = aux_ref[...]
    an_all = aux_blk / (jnp.sqrt(
        jnp.sum(aux_blk * aux_blk, axis=1, keepdims=True)) + 1e-12)

    sloss = jnp.float32(0.0)
    smax = jnp.float32(0.0)
    for rr in range(PAIRS_PER_STEP // A):
        prod = None
        for aa in range(A):
            p = rr * A + aa
            X = src_ref[p]
            xn = X / (jnp.sqrt(
                jnp.sum(X * X, axis=1, keepdims=True)) + 1e-12)
            G = lax.dot_general(
                xn, xn, (((1,), (1,)), ((), ())),
                preferred_element_type=jnp.float32,
                precision=lax.Precision.HIGHEST)
            sloss += 0.5 * (jnp.sum(jnp.abs(G)) - jnp.sum(xn * xn))
            an = an_all[p:p + 1, :]
            s = jnp.abs(lax.dot_general(
                xn, an, (((1,), (1,)), ((), ())),
                preferred_element_type=jnp.float32,
                precision=lax.Precision.HIGHEST))
            prod = s if prod is None else prod * s
        smax = jnp.maximum(smax, jnp.max(prod))

    acc_ref[0] += sloss
    acc_ref[1] = jnp.maximum(acc_ref[1], smax)

    @pl.when(i == STEPS - 1)
    def _fin():
        score = acc_ref[1]
        om = 1.0 - score
        om2 = om * om
        om4 = om2 * om2
        om8 = om4 * om4
        out_ref[0] = om8 * om2 + acc_ref[0]
        out_ref[1] = om


def kernel(table, Temp, aux_idx, source_idx):
    del Temp
    sidx = source_idx.reshape(NW, SPW, M)
    aidx = aux_idx.reshape(NW, APW)
    src_rows, aux_rows = _sc_gather_fn()(table, sidx, aidx)
    src = src_rows.reshape(RA, M, DIM)
    aux = aux_rows.reshape(RA, DIM)

    out = pl.pallas_call(
        _tc_body,
        grid=(STEPS,),
        in_specs=[
            pl.BlockSpec((PAIRS_PER_STEP, M, DIM), lambda i: (i, 0, 0)),
            pl.BlockSpec((PAIRS_PER_STEP, DIM), lambda i: (i, 0)),
        ],
        out_specs=pl.BlockSpec(memory_space=pltpu.SMEM),
        out_shape=jax.ShapeDtypeStruct((2,), jnp.float32),
        scratch_shapes=[pltpu.SMEM((2,), jnp.float32)],
    )(src, aux)
    return out


# speedup vs baseline: 1.5742x; 1.5742x over previous
import functools

import jax
import jax.numpy as jnp
from jax import lax
from jax.experimental import pallas as pl
from jax.experimental.pallas import tpu as pltpu
from jax.experimental.pallas import tpu_sc as plsc

VOCAB = 1000000
DIM = 64
R, A, M = 64, 4, 128
RA = R * A
NSRC = RA * M

NC, NS = 2, 16
NW = NC * NS
SPW = NSRC // NW // M
APW = RA // NW

PAIRS_PER_STEP = 8
STEPS = RA // PAIRS_PER_STEP

IPW = NSRC // NW
NB = 16
NBATCH = IPW // NB
G8 = VOCAB // 8
L = 16


@functools.cache
def _sc_gather_fn():
    mesh = plsc.VectorSubcoreMesh(core_axis_name="c", subcore_axis_name="s")

    @functools.partial(
        pl.kernel,
        mesh=mesh,
        out_type=[
            jax.ShapeDtypeStruct((NW * NBATCH, NB, DIM), jnp.float32),
            jax.ShapeDtypeStruct((NW, APW, DIM), jnp.float32),
        ],
        scratch_types=[
            pltpu.VMEM((IPW,), jnp.int32),
            pltpu.VMEM((L,), jnp.int32),
            pltpu.VMEM((NB, 8, DIM), jnp.float32),
            pltpu.VMEM((NB, 8, DIM), jnp.float32),
            pltpu.VMEM((NB, DIM), jnp.float32),
            pltpu.VMEM((NB, DIM), jnp.float32),
            pltpu.VMEM((L, 8, DIM), jnp.float32),
            pltpu.VMEM((L, DIM), jnp.float32),
            pltpu.SemaphoreType.DMA,
            pltpu.SemaphoreType.DMA,
        ],
        compiler_params=pltpu.CompilerParams(needs_layout_passes=False),
    )
    def _sc_gather(table_hbm, sidx_hbm, aidx_hbm, src_out, aux_out,
                   sidx_v, aidx_v, gbuf0, gbuf1, ebuf0,
                   ebuf1, agbuf, aebuf, semA, semB):
        wid = lax.axis_index("s") * NC + lax.axis_index("c")
        pltpu.sync_copy(sidx_hbm.at[wid], sidx_v)
        pltpu.sync_copy(aidx_hbm.at[pl.ds(wid * APW, APW)],
                        aidx_v.at[pl.ds(0, APW)])

        lanes = lax.iota(jnp.int32, L)

        def lane_scalar(vec, b):
            return jnp.max(jnp.where(lanes == b, vec, -1))

        av = jnp.where(lanes < APW, aidx_v[pl.ds(0, L)], 0)
        agvec = lax.shift_right_logical(av, 3)
        aovec = lax.bitwise_and(av, 7)
        acps = []
        for b in range(APW):
            g = lane_scalar(agvec, b)
            acps.append(
                pltpu.async_copy(table_hbm.at[g], agbuf.at[b], semB))
        for c in acps:
            c.wait()
        for col in range(DIM):
            cvec = jnp.full((L,), col, jnp.int32)
            x = plsc.load_gather(agbuf, [lanes, aovec, cvec])
            plsc.store_scatter(aebuf, [lanes, cvec], x)
        pltpu.sync_copy(aebuf.at[pl.ds(0, APW)], aux_out.at[wid])

        def fire(batch, buf, sem):
            svec = sidx_v[pl.ds(batch * NB, NB)]
            gvec = lax.shift_right_logical(svec, 3)
            cps = []
            for b in range(NB):
                g = lane_scalar(gvec, b)
                cps.append(pltpu.async_copy(table_hbm.at[g], buf.at[b], sem))
            return cps

        def drain_extract(batch, cps, buf, ebuf):
            for c in cps:
                c.wait()
            svec = sidx_v[pl.ds(batch * NB, NB)]
            ovec = lax.bitwise_and(svec, 7)
            for col in range(DIM):
                cvec = jnp.full((L,), col, jnp.int32)
                x = plsc.load_gather(buf, [lanes, ovec, cvec])
                plsc.store_scatter(ebuf, [lanes, cvec], x)
            pltpu.sync_copy(ebuf, src_out.at[wid * NBATCH + batch])

        def pair(u, carry):
            e, o = 2 * u, 2 * u + 1
            cps0 = fire(e, gbuf0, semA)
            cps1 = fire(o, gbuf1, semB)
            drain_extract(e, cps0, gbuf0, ebuf0)
            drain_extract(o, cps1, gbuf1, ebuf1)
            return carry
        lax.fori_loop(0, NBATCH // 2, pair, 0)

    return _sc_gather


def _tc_body(src_ref, aux_ref, out_ref, acc_ref):
    i = pl.program_id(0)

    @pl.when(i == 0)
    def _init():
        acc_ref[0] = 0.0
        acc_ref[1] = 0.0

    aux_blk = aux_ref[...]
    an_all = aux_blk / (jnp.sqrt(
        jnp.sum(aux_blk * aux_blk, axis=1, keepdims=True)) + 1e-12)

    sloss = jnp.float32(0.0)
    smax = jnp.float32(0.0)
    for rr in range(PAIRS_PER_STEP // A):
        prod = None
        for aa in range(A):
            p = rr * A + aa
            X = src_ref[p]
            xn = X / (jnp.sqrt(
                jnp.sum(X * X, axis=1, keepdims=True)) + 1e-12)
            G = lax.dot_general(
                xn, xn, (((1,), (1,)), ((), ())),
                preferred_element_type=jnp.float32,
                precision=lax.Precision.HIGHEST)
            sloss += 0.5 * (jnp.sum(jnp.abs(G)) - jnp.sum(xn * xn))
            an = an_all[p:p + 1, :]
            s = jnp.abs(lax.dot_general(
                xn, an, (((1,), (1,)), ((), ())),
                preferred_element_type=jnp.float32,
                precision=lax.Precision.HIGHEST))
            prod = s if prod is None else prod * s
        smax = jnp.maximum(smax, jnp.max(prod))

    acc_ref[0] += sloss
    acc_ref[1] = jnp.maximum(acc_ref[1], smax)

    @pl.when(i == STEPS - 1)
    def _fin():
        score = acc_ref[1]
        om = 1.0 - score
        om2 = om * om
        om4 = om2 * om2
        om8 = om4 * om4
        out_ref[0] = om8 * om2 + acc_ref[0]
        out_ref[1] = om


def kernel(table, Temp, aux_idx, source_idx):
    del Temp
    table8 = table.reshape(G8, 8, DIM)
    sidx = source_idx.reshape(NW, IPW)
    aidx = aux_idx.reshape(RA)
    src_rows, aux_rows = _sc_gather_fn()(table8, sidx, aidx)
    src = src_rows.reshape(RA, M, DIM)
    aux = aux_rows.reshape(RA, DIM)

    out = pl.pallas_call(
        _tc_body,
        grid=(STEPS,),
        in_specs=[
            pl.BlockSpec((PAIRS_PER_STEP, M, DIM), lambda i: (i, 0, 0)),
            pl.BlockSpec((PAIRS_PER_STEP, DIM), lambda i: (i, 0)),
        ],
        out_specs=pl.BlockSpec(memory_space=pltpu.SMEM),
        out_shape=jax.ShapeDtypeStruct((2,), jnp.float32),
        scratch_shapes=[pltpu.SMEM((2,), jnp.float32)],
    )(src, aux)
    return out


# speedup vs baseline: 1.5757x; 1.0010x over previous
import functools

import jax
import jax.numpy as jnp
from jax import lax
from jax.experimental import pallas as pl
from jax.experimental.pallas import tpu as pltpu
from jax.experimental.pallas import tpu_sc as plsc

VOCAB = 1000000
DIM = 64
R, A, M = 64, 4, 128
RA = R * A
NSRC = RA * M

NC, NS = 2, 16
NW = NC * NS
SPW = NSRC // NW // M
APW = RA // NW

PAIRS_PER_STEP = 8
STEPS = RA // PAIRS_PER_STEP

IPW = NSRC // NW
NB = 16
NBATCH = IPW // NB
G8 = VOCAB // 8
L = 16


@functools.cache
def _sc_gather_fn():
    mesh = plsc.VectorSubcoreMesh(core_axis_name="c", subcore_axis_name="s")

    @functools.partial(
        pl.kernel,
        mesh=mesh,
        out_type=[
            jax.ShapeDtypeStruct((NW * NBATCH, NB, DIM), jnp.float32),
            jax.ShapeDtypeStruct((NW, APW, DIM), jnp.float32),
        ],
        scratch_types=[
            pltpu.VMEM((IPW,), jnp.int32),
            pltpu.VMEM((L,), jnp.int32),
            pltpu.VMEM((NB, 8, DIM), jnp.float32),
            pltpu.VMEM((NB, 8, DIM), jnp.float32),
            pltpu.VMEM((NB, DIM), jnp.float32),
            pltpu.VMEM((NB, DIM), jnp.float32),
            pltpu.VMEM((L, 8, DIM), jnp.float32),
            pltpu.VMEM((L, DIM), jnp.float32),
            pltpu.SemaphoreType.DMA,
            pltpu.SemaphoreType.DMA,
        ],
        compiler_params=pltpu.CompilerParams(
            needs_layout_passes=False, use_tc_tiling_on_sc=True),
    )
    def _sc_gather(table_hbm, sidx_hbm, aidx_hbm, src_out, aux_out,
                   sidx_v, aidx_v, gbuf0, gbuf1, ebuf0,
                   ebuf1, agbuf, aebuf, semA, semB):
        wid = lax.axis_index("s") * NC + lax.axis_index("c")
        pltpu.sync_copy(sidx_hbm.at[wid], sidx_v)
        pltpu.sync_copy(aidx_hbm.at[pl.ds(wid * APW, APW)],
                        aidx_v.at[pl.ds(0, APW)])

        lanes = lax.iota(jnp.int32, L)

        def lane_scalar(vec, b):
            return jnp.max(jnp.where(lanes == b, vec, -1))

        av = jnp.where(lanes < APW, aidx_v[pl.ds(0, L)], 0)
        agvec = lax.shift_right_logical(av, 3)
        aovec = lax.bitwise_and(av, 7)
        acps = []
        for b in range(APW):
            g = lane_scalar(agvec, b)
            acps.append(
                pltpu.async_copy(table_hbm.at[g], agbuf.at[b], semB))
        for c in acps:
            c.wait()
        for col in range(DIM):
            cvec = jnp.full((L,), col, jnp.int32)
            x = plsc.load_gather(agbuf, [lanes, aovec, cvec])
            plsc.store_scatter(aebuf, [lanes, cvec], x)
        pltpu.sync_copy(aebuf.at[pl.ds(0, APW)], aux_out.at[wid])

        def fire(batch, buf, sem):
            svec = sidx_v[pl.ds(batch * NB, NB)]
            gvec = lax.shift_right_logical(svec, 3)
            cps = []
            for b in range(NB):
                g = lane_scalar(gvec, b)
                cps.append(pltpu.async_copy(table_hbm.at[g], buf.at[b], sem))
            return cps

        def drain_extract(batch, cps, buf, ebuf):
            for c in cps:
                c.wait()
            svec = sidx_v[pl.ds(batch * NB, NB)]
            ovec = lax.bitwise_and(svec, 7)
            for col in range(DIM):
                cvec = jnp.full((L,), col, jnp.int32)
                x = plsc.load_gather(buf, [lanes, ovec, cvec])
                plsc.store_scatter(ebuf, [lanes, cvec], x)
            pltpu.sync_copy(ebuf, src_out.at[wid * NBATCH + batch])

        def pair(u, carry):
            e, o = 2 * u, 2 * u + 1
            cps0 = fire(e, gbuf0, semA)
            cps1 = fire(o, gbuf1, semB)
            drain_extract(e, cps0, gbuf0, ebuf0)
            drain_extract(o, cps1, gbuf1, ebuf1)
            return carry
        lax.fori_loop(0, NBATCH // 2, pair, 0)

    return _sc_gather


def _tc_body(src_ref, aux_ref, out_ref, acc_ref):
    i = pl.program_id(0)

    @pl.when(i == 0)
    def _init():
        acc_ref[0] = 0.0
        acc_ref[1] = 0.0

    aux_blk = aux_ref[...]
    an_all = aux_blk / (jnp.sqrt(
        jnp.sum(aux_blk * aux_blk, axis=1, keepdims=True)) + 1e-12)

    sloss = jnp.float32(0.0)
    smax = jnp.float32(0.0)
    for rr in range(PAIRS_PER_STEP // A):
        prod = None
        for aa in range(A):
            p = rr * A + aa
            X = src_ref[p]
            xn = X / (jnp.sqrt(
                jnp.sum(X * X, axis=1, keepdims=True)) + 1e-12)
            G = lax.dot_general(
                xn, xn, (((1,), (1,)), ((), ())),
                preferred_element_type=jnp.float32,
                precision=lax.Precision.HIGHEST)
            sloss += 0.5 * (jnp.sum(jnp.abs(G)) - jnp.sum(xn * xn))
            an = an_all[p:p + 1, :]
            s = jnp.abs(lax.dot_general(
                xn, an, (((1,), (1,)), ((), ())),
                preferred_element_type=jnp.float32,
                precision=lax.Precision.HIGHEST))
            prod = s if prod is None else prod * s
        smax = jnp.maximum(smax, jnp.max(prod))

    acc_ref[0] += sloss
    acc_ref[1] = jnp.maximum(acc_ref[1], smax)

    @pl.when(i == STEPS - 1)
    def _fin():
        score = acc_ref[1]
        om = 1.0 - score
        om2 = om * om
        om4 = om2 * om2
        om8 = om4 * om4
        out_ref[0] = om8 * om2 + acc_ref[0]
        out_ref[1] = om


def kernel(table, Temp, aux_idx, source_idx):
    del Temp
    table8 = table.reshape(G8, 8, DIM)
    sidx = source_idx.reshape(NW, IPW)
    aidx = aux_idx.reshape(RA)
    src_rows, aux_rows = _sc_gather_fn()(table8, sidx, aidx)
    src = src_rows.reshape(RA, M, DIM)
    aux = aux_rows.reshape(RA, DIM)

    out = pl.pallas_call(
        _tc_body,
        grid=(STEPS,),
        in_specs=[
            pl.BlockSpec((PAIRS_PER_STEP, M, DIM), lambda i: (i, 0, 0)),
            pl.BlockSpec((PAIRS_PER_STEP, DIM), lambda i: (i, 0)),
        ],
        out_specs=pl.BlockSpec(memory_space=pltpu.SMEM),
        out_shape=jax.ShapeDtypeStruct((2,), jnp.float32),
        scratch_shapes=[pltpu.SMEM((2,), jnp.float32)],
    )(src, aux)
    return out
